# Initial kernel scaffold; baseline (speedup 1.0000x reference)
import jax
import jax.numpy as jnp
from jax.experimental import pallas as pl


def kernel(x, gate_W, gate_b, Wu, bu, Wg, bg, Wd, bd):
    raise NotImplementedError("write your pallas kernel here")


# speedup vs baseline: 5.5214x; 5.5214x over previous
import functools

import jax
import jax.numpy as jnp
from jax.experimental import pallas as pl
from jax.experimental.pallas import tpu as pltpu

NE = 16
K = 2
HD = 1024
FF = 2048
L = 4096
P = L * K
RB = 512
NB = P // RB + NE
NP = NB * RB
FB = 512
NF = FF // FB


def _ffn_body(be_ref, nv_ref, xs_ref, wu_ref, bu_ref, wg_ref, bg_ref,
              wd_ref, bd_ref, ys_ref):
    b = pl.program_id(0)
    f = pl.program_id(1)

    @pl.when(nv_ref[b] > 0)
    def _():
        xb = xs_ref[...]
        u = jnp.dot(xb, wu_ref[0], preferred_element_type=jnp.float32) + bu_ref[0]
        g = jnp.dot(xb, wg_ref[0], preferred_element_type=jnp.float32) + bg_ref[0]
        a = u * (g * jax.nn.sigmoid(g))
        y = jnp.dot(a, wd_ref[0], preferred_element_type=jnp.float32)

        @pl.when(f == 0)
        def _():
            ys_ref[...] = y + bd_ref[0, 0]

        @pl.when(f != 0)
        def _():
            ys_ref[...] = ys_ref[...] + y


def _expert_ffn(be, nv, xs, Wu, bu, Wg, bg, Wd, bd):
    grid_spec = pltpu.PrefetchScalarGridSpec(
        num_scalar_prefetch=2,
        grid=(NB, NF),
        in_specs=[
            pl.BlockSpec((RB, HD), lambda b, f, be, nv: (b, 0)),
            pl.BlockSpec((1, HD, FB), lambda b, f, be, nv: (be[b], 0, f)),
            pl.BlockSpec((1, 1, FB), lambda b, f, be, nv: (be[b], 0, f)),
            pl.BlockSpec((1, HD, FB), lambda b, f, be, nv: (be[b], 0, f)),
            pl.BlockSpec((1, 1, FB), lambda b, f, be, nv: (be[b], 0, f)),
            pl.BlockSpec((1, FB, HD), lambda b, f, be, nv: (be[b], f, 0)),
            pl.BlockSpec((1, 1, HD), lambda b, f, be, nv: (be[b], 0, 0)),
        ],
        out_specs=pl.BlockSpec((RB, HD), lambda b, f, be, nv: (b, 0)),
    )
    return pl.pallas_call(
        _ffn_body,
        grid_spec=grid_spec,
        out_shape=jax.ShapeDtypeStruct((NP, HD), jnp.float32),
        compiler_params=pltpu.CompilerParams(
            dimension_semantics=("arbitrary", "arbitrary"),
        ),
    )(be, nv, xs, Wu, bu.reshape(NE, 1, FF), Wg, bg.reshape(NE, 1, FF),
      Wd, bd.reshape(NE, 1, HD))


def kernel(x, gate_W, gate_b, Wu, bu, Wg, bg, Wd, bd):
    logits = x @ gate_W.T + gate_b
    probs = jax.nn.softmax(logits, axis=-1)
    weights, expert_ids = jax.lax.top_k(probs, K)
    e_flat = expert_ids.reshape(-1).astype(jnp.int32)
    w_flat = weights.reshape(-1)

    oh = (e_flat[:, None] == jnp.arange(NE, dtype=jnp.int32)[None, :]).astype(jnp.int32)
    counts = jnp.sum(oh, axis=0)
    rank = jnp.take_along_axis(jnp.cumsum(oh, axis=0) - oh, e_flat[:, None], axis=1)[:, 0]
    blocks_per_e = (counts + RB - 1) // RB
    bstart = jnp.cumsum(blocks_per_e) - blocks_per_e
    dest = bstart[e_flat] * RB + rank
    barange = jnp.arange(NB, dtype=jnp.int32)
    be = jnp.clip(jnp.searchsorted(bstart, barange, side="right").astype(jnp.int32) - 1, 0, NE - 1)
    nv = jnp.clip(counts[be] - (barange - bstart[be]) * RB, 0, RB).astype(jnp.int32)

    x_pairs = jnp.repeat(x, K, axis=0)
    xs = jnp.zeros((NP, HD), jnp.float32).at[dest].set(x_pairs)

    ys = _expert_ffn(be, nv, xs, Wu, bu, Wg, bg, Wd, bd)

    d = dest.reshape(L, K)
    w2 = w_flat.reshape(L, K)
    out = w2[:, 0:1] * ys[d[:, 0]] + w2[:, 1:2] * ys[d[:, 1]]
    return out
